# baseline (speedup 1.0000x reference)
import functools

import jax
import jax.numpy as jnp
from jax import lax
from jax.experimental import pallas as pl
from jax.experimental.pallas import tpu as pltpu
from jax.experimental.pallas import tpu_sc as plsc


def _sc_gather_wide(table, idx):
    V, D = table.shape
    B = idx.shape[0]
    NW = 32
    b_per_w = B // NW
    mesh = plsc.VectorSubcoreMesh(core_axis_name="c", subcore_axis_name="s")

    @functools.partial(
        pl.kernel,
        mesh=mesh,
        compiler_params=pltpu.CompilerParams(use_tc_tiling_on_sc=False),
        out_type=jax.ShapeDtypeStruct((B, 2 * D), jnp.float32),
        scratch_types=[
            pltpu.VMEM((b_per_w,), jnp.int32),
            pltpu.VMEM((b_per_w, D), jnp.float32),
            pltpu.SemaphoreType.DMA,
        ],
    )
    def gather_kernel(table_hbm, idx_hbm, out_hbm, idx_v, rows_v, sem):
        wid = lax.axis_index("s") * 2 + lax.axis_index("c")
        base = wid * b_per_w
        pltpu.sync_copy(idx_hbm.at[pl.ds(base, b_per_w)], idx_v)
        pltpu.async_copy(table_hbm.at[idx_v], rows_v, sem).wait()
        pltpu.sync_copy(rows_v, out_hbm.at[pl.ds(base, b_per_w), pl.ds(0, D)])

    return gather_kernel(table, idx)


def _proj_body(emb_ref, w_ref, b_ref, out_ref):
    out_ref[...] = (
        jnp.dot(emb_ref[:, :64], w_ref[...], preferred_element_type=jnp.float32)
        + b_ref[...]
    )


def _tc_project(emb_wide, W, b):
    B = emb_wide.shape[0]
    D, N = W.shape
    BM = 2048
    return pl.pallas_call(
        _proj_body,
        grid=(B // BM,),
        in_specs=[
            pl.BlockSpec((BM, 2 * D), lambda i: (i, 0)),
            pl.BlockSpec((D, N), lambda i: (0, 0)),
            pl.BlockSpec((1, N), lambda i: (0, 0)),
        ],
        out_specs=pl.BlockSpec((BM, N), lambda i: (i, 0)),
        out_shape=jax.ShapeDtypeStruct((B, N), jnp.float32),
    )(emb_wide, W, b.reshape(1, N))


def kernel(user_embeds, user_table, W, b):
    emb_wide = _sc_gather_wide(user_table, user_embeds)
    return _tc_project(emb_wide, W, b)


# speedup vs baseline: 1.4193x; 1.4193x over previous
import functools

import jax
import jax.numpy as jnp
from jax import lax
from jax.experimental import pallas as pl
from jax.experimental.pallas import tpu as pltpu
from jax.experimental.pallas import tpu_sc as plsc


def _sc_gather_wide(table_pad, idx):
    V, DW = table_pad.shape
    Bc = idx.shape[0]
    NW = 32
    b_per_w = Bc // NW
    mesh = plsc.VectorSubcoreMesh(core_axis_name="c", subcore_axis_name="s")

    @functools.partial(
        pl.kernel,
        mesh=mesh,
        compiler_params=pltpu.CompilerParams(use_tc_tiling_on_sc=False),
        out_type=jax.ShapeDtypeStruct((Bc, DW), jnp.float32),
        scratch_types=[
            pltpu.VMEM((b_per_w,), jnp.int32),
            pltpu.VMEM((b_per_w, DW), jnp.float32),
            pltpu.SemaphoreType.DMA,
        ],
    )
    def gather_kernel(table_hbm, idx_hbm, out_hbm, idx_v, rows_v, sem):
        wid = lax.axis_index("s") * 2 + lax.axis_index("c")
        base = wid * b_per_w
        pltpu.sync_copy(idx_hbm.at[pl.ds(base, b_per_w)], idx_v)
        pltpu.async_copy(table_hbm.at[idx_v], rows_v, sem).wait()
        pltpu.sync_copy(rows_v, out_hbm.at[pl.ds(base, b_per_w)])

    return gather_kernel(table_pad, idx)


def _proj_body(emb_ref, w_ref, b_ref, out_ref):
    out_ref[...] = (
        jnp.dot(emb_ref[:, :64], w_ref[...], preferred_element_type=jnp.float32)
        + b_ref[...]
    )


def _proj_body_aliased(emb_ref, w_ref, b_ref, prev_ref, out_ref):
    del prev_ref
    out_ref[...] = (
        jnp.dot(emb_ref[:, :64], w_ref[...], preferred_element_type=jnp.float32)
        + b_ref[...]
    )


def _tc_project_half(emb_wide, W, b2, half, prev=None):
    Bc = emb_wide.shape[0]
    D, N = W.shape
    B = 2 * Bc
    BM = 4096
    grid = (Bc // BM,)
    in_specs = [
        pl.BlockSpec((BM, 2 * D), lambda i: (i, 0)),
        pl.BlockSpec((D, N), lambda i: (0, 0)),
        pl.BlockSpec((1, N), lambda i: (0, 0)),
    ]
    args = [emb_wide, W, b2]
    kwargs = {}
    body = _proj_body
    if prev is not None:
        in_specs.append(pl.BlockSpec(memory_space=pl.ANY))
        args.append(prev)
        kwargs["input_output_aliases"] = {3: 0}
        body = _proj_body_aliased
    off = half * (Bc // BM)
    return pl.pallas_call(
        body,
        grid=grid,
        in_specs=in_specs,
        out_specs=pl.BlockSpec((BM, N), lambda i: (i + off, 0)),
        out_shape=jax.ShapeDtypeStruct((B, N), jnp.float32),
        **kwargs,
    )(*args)


def kernel(user_embeds, user_table, W, b):
    V, D = user_table.shape
    B = user_embeds.shape[0]
    pad_id = jnp.eye(D, 2 * D, dtype=user_table.dtype)
    table_pad = user_table @ pad_id
    b2 = b.reshape(1, -1)
    emb0 = _sc_gather_wide(table_pad, lax.slice(user_embeds, (0,), (B // 2,)))
    emb1 = _sc_gather_wide(table_pad, lax.slice(user_embeds, (B // 2,), (B,)))
    out0 = _tc_project_half(emb0, W, b2, half=0)
    return _tc_project_half(emb1, W, b2, half=1, prev=out0)


# speedup vs baseline: 1.5289x; 1.0772x over previous
import functools

import jax
import jax.numpy as jnp
from jax import lax
from jax.experimental import pallas as pl
from jax.experimental.pallas import tpu as pltpu
from jax.experimental.pallas import tpu_sc as plsc


def _sc_gather_wide(table_pad, idx):
    V, DW = table_pad.shape
    B = idx.shape[0]
    NW = 32
    b_per_w = B // NW
    mesh = plsc.VectorSubcoreMesh(core_axis_name="c", subcore_axis_name="s")

    @functools.partial(
        pl.kernel,
        mesh=mesh,
        compiler_params=pltpu.CompilerParams(use_tc_tiling_on_sc=False),
        out_type=jax.ShapeDtypeStruct((B, DW), jnp.float32),
        scratch_types=[
            pltpu.VMEM((b_per_w,), jnp.int32),
            pltpu.VMEM((b_per_w, DW), jnp.float32),
            pltpu.SemaphoreType.DMA,
        ],
    )
    def gather_kernel(table_hbm, idx_hbm, out_hbm, idx_v, rows_v, sem):
        wid = lax.axis_index("s") * 2 + lax.axis_index("c")
        base = wid * b_per_w
        pltpu.sync_copy(idx_hbm.at[pl.ds(base, b_per_w)], idx_v)
        pltpu.async_copy(table_hbm.at[idx_v], rows_v, sem).wait()
        pltpu.sync_copy(rows_v, out_hbm.at[pl.ds(base, b_per_w)])

    return gather_kernel(table_pad, idx)


def _proj_body(emb_ref, w_ref, b_ref, out_ref):
    out_ref[...] = (
        jnp.dot(emb_ref[:, :64], w_ref[...], preferred_element_type=jnp.float32)
        + b_ref[...]
    )


def _tc_project(emb_wide, W, b):
    B = emb_wide.shape[0]
    D, N = W.shape
    BM = 4096
    return pl.pallas_call(
        _proj_body,
        grid=(B // BM,),
        in_specs=[
            pl.BlockSpec((BM, 2 * D), lambda i: (i, 0)),
            pl.BlockSpec((D, N), lambda i: (0, 0)),
            pl.BlockSpec((1, N), lambda i: (0, 0)),
        ],
        out_specs=pl.BlockSpec((BM, N), lambda i: (i, 0)),
        out_shape=jax.ShapeDtypeStruct((B, N), jnp.float32),
    )(emb_wide, W, b.reshape(1, N))


def kernel(user_embeds, user_table, W, b):
    V, D = user_table.shape
    pad_id = jnp.eye(D, 2 * D, dtype=user_table.dtype)
    table_pad = user_table @ pad_id
    emb_wide = _sc_gather_wide(table_pad, user_embeds)
    return _tc_project(emb_wide, W, b)
